# Initial kernel scaffold; baseline (speedup 1.0000x reference)
import jax
import jax.numpy as jnp
from jax.experimental import pallas as pl


def kernel(logits, target):
    raise NotImplementedError("write your pallas kernel here")


# speedup vs baseline: 4.3867x; 4.3867x over previous
import jax
import jax.numpy as jnp
from jax.experimental import pallas as pl
from jax.experimental.pallas import tpu as pltpu

B, C, H, W = 8, 8, 512, 512
RB = 256
NK = H // RB


def _dice_body(logits_ref, target_ref, out_ref, acc_ref, stats_ref):
    b = pl.program_id(0)
    k = pl.program_id(1)

    @pl.when(k == 0)
    def _init():
        acc_ref[...] = jnp.zeros_like(acc_ref)

    x = logits_ref[0]
    t = target_ref[0]

    m = x[0]
    for c in range(1, C):
        m = jnp.maximum(m, x[c])

    pred = jnp.full(t.shape, C, jnp.int32)
    for c in range(C - 1, -1, -1):
        pred = jnp.where(x[c] == m, c, pred)

    eqpt = pred == t
    for c in range(C):
        pc = pred == c
        tc = t == c
        ic = eqpt & tc
        both = jnp.where(pc, 1.0, 0.0) + jnp.where(tc, 1.0, 0.0)
        acc_ref[pl.ds(c, 1)] += jnp.sum(both, axis=0, keepdims=True)
        acc_ref[pl.ds(C + c, 1)] += jnp.sum(
            jnp.where(ic, 1.0, 0.0), axis=0, keepdims=True)

    @pl.when(k == NK - 1)
    def _flush():
        for c in range(C):
            stats_ref[0, b, c] = jnp.sum(acc_ref[c])
            stats_ref[1, b, c] = jnp.sum(acc_ref[C + c])

    @pl.when((b == B - 1) & (k == NK - 1))
    def _final():
        total = jnp.float32(0.0)
        for bb in range(B):
            for c in range(C):
                s1 = stats_ref[0, bb, c]
                si = stats_ref[1, bb, c]
                total += 1.0 - (si + 1.0) / (s1 - si + 1.0)
        out_ref[0, 0] = total / (B * C)


def kernel(logits, target):
    out = pl.pallas_call(
        _dice_body,
        grid=(B, NK),
        in_specs=[
            pl.BlockSpec((1, C, RB, W), lambda b, k: (b, 0, k, 0)),
            pl.BlockSpec((1, RB, W), lambda b, k: (b, k, 0)),
        ],
        out_specs=pl.BlockSpec(
            (1, 1), lambda b, k: (0, 0), memory_space=pltpu.SMEM),
        out_shape=jax.ShapeDtypeStruct((1, 1), jnp.float32),
        scratch_shapes=[
            pltpu.VMEM((2 * C, W), jnp.float32),
            pltpu.SMEM((2, B, C), jnp.float32),
        ],
    )(logits, target)
    return out[0, 0]
